# Initial kernel scaffold; baseline (speedup 1.0000x reference)
import jax
import jax.numpy as jnp
from jax.experimental import pallas as pl


def kernel(x, pos, mask, edge_attr, params, edge_index):
    raise NotImplementedError("write your pallas kernel here")


# speedup vs baseline: 1.0139x; 1.0139x over previous
import jax
import jax.numpy as jnp
from jax.experimental import pallas as pl

N = 10000
E = 320000
HID = 64
NORM_FACTOR = 100.0
NORM_CONST = 1.0
COORDS_RANGE = 15.0


def _silu(v):
    return v * jax.nn.sigmoid(v)


def _matmul_bias_kernel(x_ref, w_ref, b_ref, o_ref):
    o_ref[...] = x_ref[...] @ w_ref[...] + b_ref[...]


def _emb_out(h, W, b):
    blk = 1000
    grid = N // blk
    return pl.pallas_call(
        _matmul_bias_kernel,
        grid=(grid,),
        in_specs=[
            pl.BlockSpec((blk, HID), lambda i: (i, 0)),
            pl.BlockSpec((HID, W.shape[1]), lambda i: (0, 0)),
            pl.BlockSpec((W.shape[1],), lambda i: (0,)),
        ],
        out_specs=pl.BlockSpec((blk, W.shape[1]), lambda i: (i, 0)),
        out_shape=jax.ShapeDtypeStruct((N, W.shape[1]), jnp.float32),
    )(h, W, b)


def kernel(x, pos, mask, edge_attr, params, edge_index):
    row = edge_index[0]
    col = edge_index[1]
    diff0 = pos[row] - pos[col]
    dist = jnp.sum(diff0 * diff0, axis=-1, keepdims=True)
    ea = jnp.concatenate([dist, edge_attr], axis=-1)
    W, b = params['emb_in']
    h = x @ W + b
    for blk in params['blocks']:
        diff = pos[row] - pos[col]
        radial = jnp.sum(diff * diff, axis=-1, keepdims=True)
        norm = jnp.sqrt(radial + 1e-8)
        cdiff = diff / (norm + NORM_CONST)
        for gcl in blk['gcls']:
            inp = jnp.concatenate([h[row], h[col], ea], axis=-1)
            (W1, b1), (W2, b2) = gcl['edge_mlp']
            m = _silu(_silu(inp @ W1 + b1) @ W2 + b2)
            agg = jax.ops.segment_sum(m, row, num_segments=N) / NORM_FACTOR
            (Wn1, bn1), (Wn2, bn2) = gcl['node_mlp']
            upd = _silu(jnp.concatenate([h, agg], axis=-1) @ Wn1 + bn1) @ Wn2 + bn2
            h = (h + upd) * mask
        inp = jnp.concatenate([h[row], h[col], ea], axis=-1)
        (Wc1, bc1), (Wc2, bc2), (Wc3,) = blk['coord_mlp']
        phi = _silu(_silu(inp @ Wc1 + bc1) @ Wc2 + bc2) @ Wc3
        phi = jnp.tanh(phi) * COORDS_RANGE
        trans = cdiff * phi
        cagg = jax.ops.segment_sum(trans, row, num_segments=N) / NORM_FACTOR
        pos = pos + cagg * mask
        h = h * mask
    Wo, bo = params['emb_out']
    h = _emb_out(h, Wo, bo) * mask
    return (h, pos)


# speedup vs baseline: 1.0496x; 1.0352x over previous
import jax
import jax.numpy as jnp
from jax.experimental import pallas as pl

N = 10000
E = 320000
HID = 64
IN_F = 128
OUT_F = 128
NORM_FACTOR = 100.0
NORM_CONST = 1.0
COORDS_RANGE = 15.0

NBLK = 1000
EBLK = 2560


def _silu(v):
    return v * jax.nn.sigmoid(v)


def _full(shape):
    return pl.BlockSpec(shape, lambda i: tuple(0 for _ in shape))


def _emb_in_body(x_ref, we_ref, be_ref, wr_ref, wc_ref, h_ref, a_ref, b_ref):
    h = x_ref[...] @ we_ref[...] + be_ref[...]
    h_ref[...] = h
    a_ref[...] = h @ wr_ref[...]
    b_ref[...] = h @ wc_ref[...]


def _emb_in(x, We, be, W1r, W1c):
    return pl.pallas_call(
        _emb_in_body,
        grid=(N // NBLK,),
        in_specs=[
            pl.BlockSpec((NBLK, IN_F), lambda i: (i, 0)),
            _full((IN_F, HID)), _full((HID,)),
            _full((HID, HID)), _full((HID, HID)),
        ],
        out_specs=[
            pl.BlockSpec((NBLK, HID), lambda i: (i, 0)),
            pl.BlockSpec((NBLK, HID), lambda i: (i, 0)),
            pl.BlockSpec((NBLK, HID), lambda i: (i, 0)),
        ],
        out_shape=[
            jax.ShapeDtypeStruct((N, HID), jnp.float32),
            jax.ShapeDtypeStruct((N, HID), jnp.float32),
            jax.ShapeDtypeStruct((N, HID), jnp.float32),
        ],
    )(x, We, be, W1r, W1c)


def _node_update_body(h_ref, s_ref, mask_ref, wn1h_ref, wn1a_ref, bn1_ref,
                      wn2_ref, bn2_ref, *proj_refs):
    nproj = (len(proj_refs) - 1) // 2
    h = h_ref[...]
    agg = s_ref[...] * (1.0 / NORM_FACTOR)
    u = _silu(h @ wn1h_ref[...] + agg @ wn1a_ref[...] + bn1_ref[...])
    hn = (h + u @ wn2_ref[...] + bn2_ref[...]) * mask_ref[...]
    out_refs = proj_refs[nproj:]
    out_refs[0][...] = hn
    for k in range(nproj):
        out_refs[1 + k][...] = hn @ proj_refs[k][...]


def _node_update(h, s, mask, Wn1h, Wn1a, bn1, Wn2, bn2, proj_ws):
    nproj = len(proj_ws)
    return pl.pallas_call(
        _node_update_body,
        grid=(N // NBLK,),
        in_specs=[
            pl.BlockSpec((NBLK, HID), lambda i: (i, 0)),
            pl.BlockSpec((NBLK, HID), lambda i: (i, 0)),
            pl.BlockSpec((NBLK, 1), lambda i: (i, 0)),
            _full((HID, HID)), _full((HID, HID)), _full((HID,)),
            _full((HID, HID)), _full((HID,)),
        ] + [_full((HID, HID)) for _ in range(nproj)],
        out_specs=[pl.BlockSpec((NBLK, HID), lambda i: (i, 0))
                   for _ in range(1 + nproj)],
        out_shape=[jax.ShapeDtypeStruct((N, HID), jnp.float32)
                   for _ in range(1 + nproj)],
    )(h, s, mask, Wn1h, Wn1a, bn1, Wn2, bn2, *proj_ws)


def _emb_out_body(h_ref, w_ref, b_ref, mask_ref, o_ref):
    o_ref[...] = (h_ref[...] @ w_ref[...] + b_ref[...]) * mask_ref[...]


def _emb_out(h, Wo, bo, mask):
    return pl.pallas_call(
        _emb_out_body,
        grid=(N // NBLK,),
        in_specs=[
            pl.BlockSpec((NBLK, HID), lambda i: (i, 0)),
            _full((HID, OUT_F)), _full((OUT_F,)),
            pl.BlockSpec((NBLK, 1), lambda i: (i, 0)),
        ],
        out_specs=pl.BlockSpec((NBLK, OUT_F), lambda i: (i, 0)),
        out_shape=jax.ShapeDtypeStruct((N, OUT_F), jnp.float32),
    )(h, Wo, bo, mask)


def _edge_mlp_body(pre_ref, ead_ref, w1e_ref, b1_ref, w2_ref, b2_ref, o_ref):
    z = _silu(pre_ref[...] + ead_ref[...] @ w1e_ref[...] + b1_ref[...])
    o_ref[...] = _silu(z @ w2_ref[...] + b2_ref[...])


def _edge_mlp(pre0, ead, W1e, b1, W2, b2):
    return pl.pallas_call(
        _edge_mlp_body,
        grid=(E // EBLK,),
        in_specs=[
            pl.BlockSpec((EBLK, HID), lambda i: (i, 0)),
            pl.BlockSpec((EBLK, 17), lambda i: (i, 0)),
            _full((17, HID)), _full((HID,)),
            _full((HID, HID)), _full((HID,)),
        ],
        out_specs=pl.BlockSpec((EBLK, HID), lambda i: (i, 0)),
        out_shape=jax.ShapeDtypeStruct((E, HID), jnp.float32),
    )(pre0, ead, W1e, b1, W2, b2)


def _coord_mlp_body(pre_ref, ead_ref, diff_ref, w1e_ref, b1_ref, w2_ref,
                    b2_ref, w3_ref, o_ref):
    z = _silu(pre_ref[...] + ead_ref[...] @ w1e_ref[...] + b1_ref[...])
    z = _silu(z @ w2_ref[...] + b2_ref[...])
    phi = jnp.tanh(z @ w3_ref[...]) * COORDS_RANGE
    d = diff_ref[...]
    radial = jnp.sum(d * d, axis=-1, keepdims=True)
    scale = phi / (jnp.sqrt(radial + 1e-8) + NORM_CONST)
    o_ref[...] = d * scale


def _coord_mlp(pre0, ead, diff, Wc1e, bc1, Wc2, bc2, Wc3):
    return pl.pallas_call(
        _coord_mlp_body,
        grid=(E // EBLK,),
        in_specs=[
            pl.BlockSpec((EBLK, HID), lambda i: (i, 0)),
            pl.BlockSpec((EBLK, 17), lambda i: (i, 0)),
            pl.BlockSpec((EBLK, 3), lambda i: (i, 0)),
            _full((17, HID)), _full((HID,)),
            _full((HID, HID)), _full((HID,)),
            _full((HID, 1)),
        ],
        out_specs=pl.BlockSpec((EBLK, 3), lambda i: (i, 0)),
        out_shape=jax.ShapeDtypeStruct((E, 3), jnp.float32),
    )(pre0, ead, diff, Wc1e, bc1, Wc2, bc2, Wc3)


def _combine(A, B, row, col):
    return A[row] + B[col]


def _segsum(vals, row):
    return jax.ops.segment_sum(vals, row, num_segments=N)


def _split_w1(W1):
    return W1[:HID], W1[HID:2 * HID], W1[2 * HID:]


def kernel(x, pos, mask, edge_attr, params, edge_index):
    row = edge_index[0]
    col = edge_index[1]
    blocks = params['blocks']

    diff0 = pos[row] - pos[col]
    dist = jnp.sum(diff0 * diff0, axis=-1, keepdims=True)
    ead = jnp.concatenate([dist, edge_attr], axis=-1)

    gcl0 = blocks[0]['gcls'][0]
    (W1_0, b1_0), (W2_0, b2_0) = gcl0['edge_mlp']
    W1r0, W1c0, W1e0 = _split_w1(W1_0)
    We, be = params['emb_in']
    h, A, B = _emb_in(x, We, be, W1r0, W1c0)

    for bi, blk in enumerate(blocks):
        gcl = blk['gcls'][0]
        (W1, b1), (W2, b2) = gcl['edge_mlp']
        W1r, W1c, W1e = _split_w1(W1)
        if bi > 0:
            pass
        pre0 = _combine(A, B, row, col)
        m = _edge_mlp(pre0, ead, W1e, b1, W2, b2)
        s = _segsum(m, row)

        (Wn1, bn1), (Wn2, bn2) = gcl['node_mlp']
        Wn1h, Wn1a = Wn1[:HID], Wn1[HID:]
        (Wc1, bc1), (Wc2, bc2), (Wc3,) = blk['coord_mlp']
        Wc1r, Wc1c, Wc1e = _split_w1(Wc1)
        proj_ws = [Wc1r, Wc1c]
        if bi + 1 < len(blocks):
            gcl_n = blocks[bi + 1]['gcls'][0]
            W1n = gcl_n['edge_mlp'][0][0]
            W1rn, W1cn, _ = _split_w1(W1n)
            proj_ws += [W1rn, W1cn]
        outs = _node_update(h, s, mask, Wn1h, Wn1a, bn1, Wn2, bn2, proj_ws)
        h, Ac, Bc = outs[0], outs[1], outs[2]
        if bi + 1 < len(blocks):
            A, B = outs[3], outs[4]

        diff = pos[row] - pos[col] if bi > 0 else diff0
        pre0c = _combine(Ac, Bc, row, col)
        trans = _coord_mlp(pre0c, ead, diff, Wc1e, bc1, Wc2, bc2, Wc3)
        cagg = _segsum(trans, row)
        pos = pos + (cagg * (1.0 / NORM_FACTOR)) * mask
        h = h * mask

    Wo, bo = params['emb_out']
    return (_emb_out(h, Wo, bo, mask), pos)


# speedup vs baseline: 1.4956x; 1.4249x over previous
import functools

import jax
import jax.numpy as jnp
from jax import lax
from jax.experimental import pallas as pl
from jax.experimental.pallas import tpu as pltpu
from jax.experimental.pallas import tpu_sc as plsc

N = 10000
E = 320000
HID = 64
IN_F = 128
OUT_F = 128
NORM_FACTOR = 100.0
NORM_CONST = 1.0
COORDS_RANGE = 15.0

NBLK = 1000
EBLK = 2560


def _silu(v):
    return v * jax.nn.sigmoid(v)


def _full(shape):
    return pl.BlockSpec(shape, lambda i: tuple(0 for _ in shape))


def _emb_in_body(x_ref, we_ref, be_ref, wr_ref, wc_ref, h_ref, a_ref, b_ref):
    h = x_ref[...] @ we_ref[...] + be_ref[...]
    h_ref[...] = h
    a_ref[...] = h @ wr_ref[...]
    b_ref[...] = h @ wc_ref[...]


def _emb_in(x, We, be, W1r, W1c):
    return pl.pallas_call(
        _emb_in_body,
        grid=(N // NBLK,),
        in_specs=[
            pl.BlockSpec((NBLK, IN_F), lambda i: (i, 0)),
            _full((IN_F, HID)), _full((HID,)),
            _full((HID, HID)), _full((HID, HID)),
        ],
        out_specs=[
            pl.BlockSpec((NBLK, HID), lambda i: (i, 0)),
            pl.BlockSpec((NBLK, HID), lambda i: (i, 0)),
            pl.BlockSpec((NBLK, HID), lambda i: (i, 0)),
        ],
        out_shape=[
            jax.ShapeDtypeStruct((N, HID), jnp.float32),
            jax.ShapeDtypeStruct((N, HID), jnp.float32),
            jax.ShapeDtypeStruct((N, HID), jnp.float32),
        ],
    )(x, We, be, W1r, W1c)


def _node_update_body(h_ref, s_ref, mask_ref, wn1h_ref, wn1a_ref, bn1_ref,
                      wn2_ref, bn2_ref, *proj_refs):
    nproj = (len(proj_refs) - 1) // 2
    h = h_ref[...]
    agg = s_ref[...] * (1.0 / NORM_FACTOR)
    u = _silu(h @ wn1h_ref[...] + agg @ wn1a_ref[...] + bn1_ref[...])
    hn = (h + u @ wn2_ref[...] + bn2_ref[...]) * mask_ref[...]
    out_refs = proj_refs[nproj:]
    out_refs[0][...] = hn
    for k in range(nproj):
        out_refs[1 + k][...] = hn @ proj_refs[k][...]


def _node_update(h, s, mask, Wn1h, Wn1a, bn1, Wn2, bn2, proj_ws):
    nproj = len(proj_ws)
    return pl.pallas_call(
        _node_update_body,
        grid=(N // NBLK,),
        in_specs=[
            pl.BlockSpec((NBLK, HID), lambda i: (i, 0)),
            pl.BlockSpec((NBLK, HID), lambda i: (i, 0)),
            pl.BlockSpec((NBLK, 1), lambda i: (i, 0)),
            _full((HID, HID)), _full((HID, HID)), _full((HID,)),
            _full((HID, HID)), _full((HID,)),
        ] + [_full((HID, HID)) for _ in range(nproj)],
        out_specs=[pl.BlockSpec((NBLK, HID), lambda i: (i, 0))
                   for _ in range(1 + nproj)],
        out_shape=[jax.ShapeDtypeStruct((N, HID), jnp.float32)
                   for _ in range(1 + nproj)],
    )(h, s, mask, Wn1h, Wn1a, bn1, Wn2, bn2, *proj_ws)


def _emb_out_body(h_ref, w_ref, b_ref, mask_ref, o_ref):
    o_ref[...] = (h_ref[...] @ w_ref[...] + b_ref[...]) * mask_ref[...]


def _emb_out(h, Wo, bo, mask):
    return pl.pallas_call(
        _emb_out_body,
        grid=(N // NBLK,),
        in_specs=[
            pl.BlockSpec((NBLK, HID), lambda i: (i, 0)),
            _full((HID, OUT_F)), _full((OUT_F,)),
            pl.BlockSpec((NBLK, 1), lambda i: (i, 0)),
        ],
        out_specs=pl.BlockSpec((NBLK, OUT_F), lambda i: (i, 0)),
        out_shape=jax.ShapeDtypeStruct((N, OUT_F), jnp.float32),
    )(h, Wo, bo, mask)


def _edge_mlp_body(prea_ref, preb_ref, ead_ref, w1e_ref, b1_ref, w2_ref,
                   b2_ref, o_ref):
    pre = prea_ref[...] + preb_ref[...]
    z = _silu(pre + ead_ref[...] @ w1e_ref[...] + b1_ref[...])
    o_ref[...] = _silu(z @ w2_ref[...] + b2_ref[...])


def _edge_mlp(preA, preB, ead, W1e, b1, W2, b2):
    return pl.pallas_call(
        _edge_mlp_body,
        grid=(E // EBLK,),
        in_specs=[
            pl.BlockSpec((EBLK, HID), lambda i: (i, 0)),
            pl.BlockSpec((EBLK, HID), lambda i: (i, 0)),
            pl.BlockSpec((EBLK, 17), lambda i: (i, 0)),
            _full((17, HID)), _full((HID,)),
            _full((HID, HID)), _full((HID,)),
        ],
        out_specs=pl.BlockSpec((EBLK, HID), lambda i: (i, 0)),
        out_shape=jax.ShapeDtypeStruct((E, HID), jnp.float32),
    )(preA, preB, ead, W1e, b1, W2, b2)


def _coord_mlp_body(prea_ref, preb_ref, ead_ref, diff_ref, w1e_ref, b1_ref,
                    w2_ref, b2_ref, w3_ref, o_ref):
    pre = prea_ref[...] + preb_ref[...]
    z = _silu(pre + ead_ref[...] @ w1e_ref[...] + b1_ref[...])
    z = _silu(z @ w2_ref[...] + b2_ref[...])
    phi = jnp.tanh(z @ w3_ref[...]) * COORDS_RANGE
    d = diff_ref[...]
    radial = jnp.sum(d * d, axis=-1, keepdims=True)
    scale = phi / (jnp.sqrt(radial + 1e-8) + NORM_CONST)
    o_ref[...] = d * scale


def _coord_mlp(preA, preB, ead, diff, Wc1e, bc1, Wc2, bc2, Wc3):
    return pl.pallas_call(
        _coord_mlp_body,
        grid=(E // EBLK,),
        in_specs=[
            pl.BlockSpec((EBLK, HID), lambda i: (i, 0)),
            pl.BlockSpec((EBLK, HID), lambda i: (i, 0)),
            pl.BlockSpec((EBLK, 17), lambda i: (i, 0)),
            pl.BlockSpec((EBLK, 3), lambda i: (i, 0)),
            _full((17, HID)), _full((HID,)),
            _full((HID, HID)), _full((HID,)),
            _full((HID, 1)),
        ],
        out_specs=pl.BlockSpec((EBLK, 3), lambda i: (i, 0)),
        out_shape=jax.ShapeDtypeStruct((E, 3), jnp.float32),
    )(preA, preB, ead, diff, Wc1e, bc1, Wc2, bc2, Wc3)


NW = 32
EPW = E // NW
CCH = 1000


def _sc_mesh():
    return plsc.VectorSubcoreMesh(core_axis_name="c", subcore_axis_name="s")


def _combine(A, B, row, col):
    @functools.partial(
        pl.kernel,
        out_type=(jax.ShapeDtypeStruct((E, HID), jnp.float32),
                  jax.ShapeDtypeStruct((E, HID), jnp.float32)),
        mesh=_sc_mesh(),
        scratch_types=[
            pltpu.VMEM((CCH,), jnp.int32),
            pltpu.VMEM((CCH,), jnp.int32),
            pltpu.VMEM((CCH, HID), jnp.float32),
            pltpu.VMEM((CCH, HID), jnp.float32),
            pltpu.SemaphoreType.DMA,
            pltpu.SemaphoreType.DMA,
        ],
        compiler_params=pltpu.CompilerParams(use_tc_tiling_on_sc=False),
    )
    def k(a_hbm, b_hbm, row_hbm, col_hbm, oa_hbm, ob_hbm,
          rid, cid, bufa, bufb, s1, s2):
        wid = lax.axis_index("s") * 2 + lax.axis_index("c")
        base = wid * EPW

        def body(j, carry):
            off = base + j * CCH
            pltpu.sync_copy(row_hbm.at[pl.ds(off, CCH)], rid)
            pltpu.sync_copy(col_hbm.at[pl.ds(off, CCH)], cid)
            ca = pltpu.async_copy(a_hbm.at[rid], bufa, s1)
            cb = pltpu.async_copy(b_hbm.at[cid], bufb, s2)
            ca.wait()
            cb.wait()
            pltpu.sync_copy(bufa, oa_hbm.at[pl.ds(off, CCH)])
            pltpu.sync_copy(bufb, ob_hbm.at[pl.ds(off, CCH)])
            return carry

        lax.fori_loop(0, EPW // CCH, body, 0)

    return k(A, B, row, col)


def _segsum(vals, row):
    return jax.ops.segment_sum(vals, row, num_segments=N)


def _split_w1(W1):
    return W1[:HID], W1[HID:2 * HID], W1[2 * HID:]


def kernel(x, pos, mask, edge_attr, params, edge_index):
    row = edge_index[0]
    col = edge_index[1]
    blocks = params['blocks']

    diff0 = pos[row] - pos[col]
    dist = jnp.sum(diff0 * diff0, axis=-1, keepdims=True)
    ead = jnp.concatenate([dist, edge_attr], axis=-1)

    gcl0 = blocks[0]['gcls'][0]
    (W1_0, b1_0), (W2_0, b2_0) = gcl0['edge_mlp']
    W1r0, W1c0, W1e0 = _split_w1(W1_0)
    We, be = params['emb_in']
    h, A, B = _emb_in(x, We, be, W1r0, W1c0)

    for bi, blk in enumerate(blocks):
        gcl = blk['gcls'][0]
        (W1, b1), (W2, b2) = gcl['edge_mlp']
        W1r, W1c, W1e = _split_w1(W1)
        preA, preB = _combine(A, B, row, col)
        m = _edge_mlp(preA, preB, ead, W1e, b1, W2, b2)
        s = _segsum(m, row)

        (Wn1, bn1), (Wn2, bn2) = gcl['node_mlp']
        Wn1h, Wn1a = Wn1[:HID], Wn1[HID:]
        (Wc1, bc1), (Wc2, bc2), (Wc3,) = blk['coord_mlp']
        Wc1r, Wc1c, Wc1e = _split_w1(Wc1)
        proj_ws = [Wc1r, Wc1c]
        if bi + 1 < len(blocks):
            gcl_n = blocks[bi + 1]['gcls'][0]
            W1n = gcl_n['edge_mlp'][0][0]
            W1rn, W1cn, _ = _split_w1(W1n)
            proj_ws += [W1rn, W1cn]
        outs = _node_update(h, s, mask, Wn1h, Wn1a, bn1, Wn2, bn2, proj_ws)
        h, Ac, Bc = outs[0], outs[1], outs[2]
        if bi + 1 < len(blocks):
            A, B = outs[3], outs[4]

        diff = pos[row] - pos[col] if bi > 0 else diff0
        pcA, pcB = _combine(Ac, Bc, row, col)
        trans = _coord_mlp(pcA, pcB, ead, diff, Wc1e, bc1, Wc2, bc2, Wc3)
        cagg = _segsum(trans, row)
        pos = pos + (cagg * (1.0 / NORM_FACTOR)) * mask
        h = h * mask

    Wo, bo = params['emb_out']
    return (_emb_out(h, Wo, bo, mask), pos)


# speedup vs baseline: 1.8827x; 1.2588x over previous
import functools

import jax
import jax.numpy as jnp
from jax import lax
from jax.experimental import pallas as pl
from jax.experimental.pallas import tpu as pltpu
from jax.experimental.pallas import tpu_sc as plsc

N = 10000
E = 320000
HID = 64
IN_F = 128
OUT_F = 128
NORM_FACTOR = 100.0
NORM_CONST = 1.0
COORDS_RANGE = 15.0

NBLK = 1000
EBLK = 2560


def _silu(v):
    return v * jax.nn.sigmoid(v)


def _full(shape):
    return pl.BlockSpec(shape, lambda i: tuple(0 for _ in shape))


def _emb_in_body(x_ref, we_ref, be_ref, wr_ref, wc_ref, h_ref, a_ref, b_ref):
    h = x_ref[...] @ we_ref[...] + be_ref[...]
    h_ref[...] = h
    a_ref[...] = h @ wr_ref[...]
    b_ref[...] = h @ wc_ref[...]


def _emb_in(x, We, be, W1r, W1c):
    return pl.pallas_call(
        _emb_in_body,
        grid=(N // NBLK,),
        in_specs=[
            pl.BlockSpec((NBLK, IN_F), lambda i: (i, 0)),
            _full((IN_F, HID)), _full((HID,)),
            _full((HID, HID)), _full((HID, HID)),
        ],
        out_specs=[
            pl.BlockSpec((NBLK, HID), lambda i: (i, 0)),
            pl.BlockSpec((NBLK, HID), lambda i: (i, 0)),
            pl.BlockSpec((NBLK, HID), lambda i: (i, 0)),
        ],
        out_shape=[
            jax.ShapeDtypeStruct((N, HID), jnp.float32),
            jax.ShapeDtypeStruct((N, HID), jnp.float32),
            jax.ShapeDtypeStruct((N, HID), jnp.float32),
        ],
    )(x, We, be, W1r, W1c)


def _node_update_body(h_ref, s0_ref, s1_ref, mask_ref, wn1h_ref, wn1a_ref,
                      bn1_ref, wn2_ref, bn2_ref, *proj_refs):
    nproj = (len(proj_refs) - 1) // 2
    h = h_ref[...]
    agg = (s0_ref[...] + s1_ref[...]) * (1.0 / NORM_FACTOR)
    u = _silu(h @ wn1h_ref[...] + agg @ wn1a_ref[...] + bn1_ref[...])
    hn = (h + u @ wn2_ref[...] + bn2_ref[...]) * mask_ref[...]
    out_refs = proj_refs[nproj:]
    out_refs[0][...] = hn
    for k in range(nproj):
        out_refs[1 + k][...] = hn @ proj_refs[k][...]


def _node_update(h, s2n, mask, Wn1h, Wn1a, bn1, Wn2, bn2, proj_ws):
    nproj = len(proj_ws)
    return pl.pallas_call(
        _node_update_body,
        grid=(N // NBLK,),
        in_specs=[
            pl.BlockSpec((NBLK, HID), lambda i: (i, 0)),
            pl.BlockSpec((NBLK, HID), lambda i: (i, 0)),
            pl.BlockSpec((NBLK, HID), lambda i: (i + N // NBLK, 0)),
            pl.BlockSpec((NBLK, 1), lambda i: (i, 0)),
            _full((HID, HID)), _full((HID, HID)), _full((HID,)),
            _full((HID, HID)), _full((HID,)),
        ] + [_full((HID, HID)) for _ in range(nproj)],
        out_specs=[pl.BlockSpec((NBLK, HID), lambda i: (i, 0))
                   for _ in range(1 + nproj)],
        out_shape=[jax.ShapeDtypeStruct((N, HID), jnp.float32)
                   for _ in range(1 + nproj)],
    )(h, s2n, s2n, mask, Wn1h, Wn1a, bn1, Wn2, bn2, *proj_ws)


def _emb_out_body(h_ref, w_ref, b_ref, mask_ref, o_ref):
    o_ref[...] = (h_ref[...] @ w_ref[...] + b_ref[...]) * mask_ref[...]


def _emb_out(h, Wo, bo, mask):
    return pl.pallas_call(
        _emb_out_body,
        grid=(N // NBLK,),
        in_specs=[
            pl.BlockSpec((NBLK, HID), lambda i: (i, 0)),
            _full((HID, OUT_F)), _full((OUT_F,)),
            pl.BlockSpec((NBLK, 1), lambda i: (i, 0)),
        ],
        out_specs=pl.BlockSpec((NBLK, OUT_F), lambda i: (i, 0)),
        out_shape=jax.ShapeDtypeStruct((N, OUT_F), jnp.float32),
    )(h, Wo, bo, mask)


def _edge_mlp_body(prea_ref, preb_ref, ead_ref, w1e_ref, b1_ref, w2_ref,
                   b2_ref, o_ref):
    pre = prea_ref[...] + preb_ref[...]
    z = _silu(pre + ead_ref[...] @ w1e_ref[...] + b1_ref[...])
    o_ref[...] = _silu(z @ w2_ref[...] + b2_ref[...])


def _edge_mlp(preA, preB, ead, W1e, b1, W2, b2):
    return pl.pallas_call(
        _edge_mlp_body,
        grid=(E // EBLK,),
        in_specs=[
            pl.BlockSpec((EBLK, HID), lambda i: (i, 0)),
            pl.BlockSpec((EBLK, HID), lambda i: (i, 0)),
            pl.BlockSpec((EBLK, 17), lambda i: (i, 0)),
            _full((17, HID)), _full((HID,)),
            _full((HID, HID)), _full((HID,)),
        ],
        out_specs=pl.BlockSpec((EBLK, HID), lambda i: (i, 0)),
        out_shape=jax.ShapeDtypeStruct((E, HID), jnp.float32),
    )(preA, preB, ead, W1e, b1, W2, b2)


def _coord_mlp_body(prea_ref, preb_ref, ead_ref, diff_ref, w1e_ref, b1_ref,
                    w2_ref, b2_ref, w3_ref, o_ref):
    pre = prea_ref[...] + preb_ref[...]
    z = _silu(pre + ead_ref[...] @ w1e_ref[...] + b1_ref[...])
    z = _silu(z @ w2_ref[...] + b2_ref[...])
    phi = jnp.tanh(z @ w3_ref[...]) * COORDS_RANGE
    d = diff_ref[...]
    radial = jnp.sum(d * d, axis=-1, keepdims=True)
    scale = phi / (jnp.sqrt(radial + 1e-8) + NORM_CONST)
    o_ref[...] = jnp.concatenate(
        [d * scale, jnp.zeros((d.shape[0], 13), jnp.float32)], axis=1)


def _coord_mlp(preA, preB, ead, diff, Wc1e, bc1, Wc2, bc2, Wc3):
    return pl.pallas_call(
        _coord_mlp_body,
        grid=(E // EBLK,),
        in_specs=[
            pl.BlockSpec((EBLK, HID), lambda i: (i, 0)),
            pl.BlockSpec((EBLK, HID), lambda i: (i, 0)),
            pl.BlockSpec((EBLK, 17), lambda i: (i, 0)),
            pl.BlockSpec((EBLK, 3), lambda i: (i, 0)),
            _full((17, HID)), _full((HID,)),
            _full((HID, HID)), _full((HID,)),
            _full((HID, 1)),
        ],
        out_specs=pl.BlockSpec((EBLK, 16), lambda i: (i, 0)),
        out_shape=jax.ShapeDtypeStruct((E, 16), jnp.float32),
    )(preA, preB, ead, diff, Wc1e, bc1, Wc2, bc2, Wc3)


NW = 32
EPW = E // NW
CCH = 1000


def _sc_mesh():
    return plsc.VectorSubcoreMesh(core_axis_name="c", subcore_axis_name="s")


def _combine(A, B, row, col):
    @functools.partial(
        pl.kernel,
        out_type=(jax.ShapeDtypeStruct((E, HID), jnp.float32),
                  jax.ShapeDtypeStruct((E, HID), jnp.float32)),
        mesh=_sc_mesh(),
        scratch_types=[
            pltpu.VMEM((CCH,), jnp.int32),
            pltpu.VMEM((CCH,), jnp.int32),
            pltpu.VMEM((CCH, HID), jnp.float32),
            pltpu.VMEM((CCH, HID), jnp.float32),
            pltpu.SemaphoreType.DMA,
            pltpu.SemaphoreType.DMA,
        ],
        compiler_params=pltpu.CompilerParams(use_tc_tiling_on_sc=False),
    )
    def k(a_hbm, b_hbm, row_hbm, col_hbm, oa_hbm, ob_hbm,
          rid, cid, bufa, bufb, s1, s2):
        wid = lax.axis_index("s") * 2 + lax.axis_index("c")
        base = wid * EPW

        def body(j, carry):
            off = base + j * CCH
            pltpu.sync_copy(row_hbm.at[pl.ds(off, CCH)], rid)
            pltpu.sync_copy(col_hbm.at[pl.ds(off, CCH)], cid)
            ca = pltpu.async_copy(a_hbm.at[rid], bufa, s1)
            cb = pltpu.async_copy(b_hbm.at[cid], bufb, s2)
            ca.wait()
            cb.wait()
            pltpu.sync_copy(bufa, oa_hbm.at[pl.ds(off, CCH)])
            pltpu.sync_copy(bufb, ob_hbm.at[pl.ds(off, CCH)])
            return carry

        lax.fori_loop(0, EPW // CCH, body, 0)

    return k(A, B, row, col)


def _segsum(vals, row, K, zeros):
    npr = N // 16

    @functools.partial(
        pl.kernel,
        out_type=jax.ShapeDtypeStruct((2 * N, K), jnp.float32),
        mesh=_sc_mesh(),
        scratch_types=[
            pltpu.VMEM((CCH,), jnp.int32),
            pltpu.VMEM((CCH, K), jnp.float32),
            pltpu.VMEM_SHARED((N, K), jnp.float32),
        ],
        compiler_params=pltpu.CompilerParams(use_tc_tiling_on_sc=False),
    )
    def k(vals_hbm, row_hbm, zeros_hbm, out_hbm, rid, vbuf, acc):
        ci = lax.axis_index("c")
        sid = lax.axis_index("s")
        wid = sid * 2 + ci
        pltpu.sync_copy(zeros_hbm.at[pl.ds(sid * npr, npr)],
                        acc.at[pl.ds(sid * npr, npr)])
        plsc.subcore_barrier()
        base = wid * EPW

        def body(j, carry):
            off = base + j * CCH
            pltpu.sync_copy(row_hbm.at[pl.ds(off, CCH)], rid)
            pltpu.sync_copy(vals_hbm.at[pl.ds(off, CCH)], vbuf)
            pltpu.sync_copy(vbuf, acc.at[rid], add=True)
            return carry

        lax.fori_loop(0, EPW // CCH, body, 0)
        plsc.subcore_barrier()
        pltpu.sync_copy(acc.at[pl.ds(sid * npr, npr)],
                        out_hbm.at[pl.ds(ci * N + sid * npr, npr)])

    return k(vals, row, zeros)


def _split_w1(W1):
    return W1[:HID], W1[HID:2 * HID], W1[2 * HID:]


def kernel(x, pos, mask, edge_attr, params, edge_index):
    row = edge_index[0]
    col = edge_index[1]
    blocks = params['blocks']

    diff0 = pos[row] - pos[col]
    dist = jnp.sum(diff0 * diff0, axis=-1, keepdims=True)
    ead = jnp.concatenate([dist, edge_attr], axis=-1)

    gcl0 = blocks[0]['gcls'][0]
    (W1_0, b1_0), (W2_0, b2_0) = gcl0['edge_mlp']
    W1r0, W1c0, W1e0 = _split_w1(W1_0)
    We, be = params['emb_in']
    h, A, B = _emb_in(x, We, be, W1r0, W1c0)
    zeros64 = jnp.zeros((N, HID), jnp.float32)
    zeros16 = jnp.zeros((N, 16), jnp.float32)

    for bi, blk in enumerate(blocks):
        gcl = blk['gcls'][0]
        (W1, b1), (W2, b2) = gcl['edge_mlp']
        W1r, W1c, W1e = _split_w1(W1)
        preA, preB = _combine(A, B, row, col)
        m = _edge_mlp(preA, preB, ead, W1e, b1, W2, b2)
        s2n = _segsum(m, row, HID, zeros64)

        (Wn1, bn1), (Wn2, bn2) = gcl['node_mlp']
        Wn1h, Wn1a = Wn1[:HID], Wn1[HID:]
        (Wc1, bc1), (Wc2, bc2), (Wc3,) = blk['coord_mlp']
        Wc1r, Wc1c, Wc1e = _split_w1(Wc1)
        proj_ws = [Wc1r, Wc1c]
        if bi + 1 < len(blocks):
            gcl_n = blocks[bi + 1]['gcls'][0]
            W1n = gcl_n['edge_mlp'][0][0]
            W1rn, W1cn, _ = _split_w1(W1n)
            proj_ws += [W1rn, W1cn]
        outs = _node_update(h, s2n, mask, Wn1h, Wn1a, bn1, Wn2, bn2, proj_ws)
        h, Ac, Bc = outs[0], outs[1], outs[2]
        if bi + 1 < len(blocks):
            A, B = outs[3], outs[4]

        diff = pos[row] - pos[col] if bi > 0 else diff0
        pcA, pcB = _combine(Ac, Bc, row, col)
        trans = _coord_mlp(pcA, pcB, ead, diff, Wc1e, bc1, Wc2, bc2, Wc3)
        c2n = _segsum(trans, row, 16, zeros16)
        cagg = c2n[:N, :3] + c2n[N:, :3]
        pos = pos + (cagg * (1.0 / NORM_FACTOR)) * mask
        h = h * mask

    Wo, bo = params['emb_out']
    return (_emb_out(h, Wo, bo, mask), pos)


# speedup vs baseline: 2.9986x; 1.5927x over previous
import functools

import jax
import jax.numpy as jnp
from jax import lax
from jax.experimental import pallas as pl
from jax.experimental.pallas import tpu as pltpu
from jax.experimental.pallas import tpu_sc as plsc

N = 10000
E = 320000
HID = 64
IN_F = 128
OUT_F = 128
NORM_FACTOR = 100.0
NORM_CONST = 1.0
COORDS_RANGE = 15.0

NBLK = 1000
EBLK = 2560


def _silu(v):
    return v * jax.nn.sigmoid(v)


def _full(shape):
    return pl.BlockSpec(shape, lambda i: tuple(0 for _ in shape))


def _emb_in_body(x_ref, we_ref, be_ref, wr_ref, wc_ref, h_ref, a_ref, b_ref):
    h = x_ref[...] @ we_ref[...] + be_ref[...]
    h_ref[...] = h
    a_ref[...] = h @ wr_ref[...]
    b_ref[...] = h @ wc_ref[...]


def _emb_in(x, We, be, W1r, W1c):
    return pl.pallas_call(
        _emb_in_body,
        grid=(N // NBLK,),
        in_specs=[
            pl.BlockSpec((NBLK, IN_F), lambda i: (i, 0)),
            _full((IN_F, HID)), _full((HID,)),
            _full((HID, HID)), _full((HID, HID)),
        ],
        out_specs=[
            pl.BlockSpec((NBLK, HID), lambda i: (i, 0)),
            pl.BlockSpec((NBLK, HID), lambda i: (i, 0)),
            pl.BlockSpec((NBLK, HID), lambda i: (i, 0)),
        ],
        out_shape=[
            jax.ShapeDtypeStruct((N, HID), jnp.float32),
            jax.ShapeDtypeStruct((N, HID), jnp.float32),
            jax.ShapeDtypeStruct((N, HID), jnp.float32),
        ],
    )(x, We, be, W1r, W1c)


def _node_update_body(h_ref, s0_ref, s1_ref, mask_ref, wn1h_ref, wn1a_ref,
                      bn1_ref, wn2_ref, bn2_ref, *proj_refs):
    nproj = (len(proj_refs) - 1) // 2
    h = h_ref[...]
    agg = (s0_ref[...] + s1_ref[...]) * (1.0 / NORM_FACTOR)
    u = _silu(h @ wn1h_ref[...] + agg @ wn1a_ref[...] + bn1_ref[...])
    hn = (h + u @ wn2_ref[...] + bn2_ref[...]) * mask_ref[...]
    out_refs = proj_refs[nproj:]
    out_refs[0][...] = hn
    for k in range(nproj):
        out_refs[1 + k][...] = hn @ proj_refs[k][...]


def _node_update(h, s2n, mask, Wn1h, Wn1a, bn1, Wn2, bn2, proj_ws):
    nproj = len(proj_ws)
    return pl.pallas_call(
        _node_update_body,
        grid=(N // NBLK,),
        in_specs=[
            pl.BlockSpec((NBLK, HID), lambda i: (i, 0)),
            pl.BlockSpec((NBLK, HID), lambda i: (i, 0)),
            pl.BlockSpec((NBLK, HID), lambda i: (i + N // NBLK, 0)),
            pl.BlockSpec((NBLK, 1), lambda i: (i, 0)),
            _full((HID, HID)), _full((HID, HID)), _full((HID,)),
            _full((HID, HID)), _full((HID,)),
        ] + [_full((HID, HID)) for _ in range(nproj)],
        out_specs=[pl.BlockSpec((NBLK, HID), lambda i: (i, 0))
                   for _ in range(1 + nproj)],
        out_shape=[jax.ShapeDtypeStruct((N, HID), jnp.float32)
                   for _ in range(1 + nproj)],
    )(h, s2n, s2n, mask, Wn1h, Wn1a, bn1, Wn2, bn2, *proj_ws)


def _emb_out_body(h_ref, w_ref, b_ref, mask_ref, o_ref):
    o_ref[...] = (h_ref[...] @ w_ref[...] + b_ref[...]) * mask_ref[...]


def _emb_out(h, Wo, bo, mask):
    return pl.pallas_call(
        _emb_out_body,
        grid=(N // NBLK,),
        in_specs=[
            pl.BlockSpec((NBLK, HID), lambda i: (i, 0)),
            _full((HID, OUT_F)), _full((OUT_F,)),
            pl.BlockSpec((NBLK, 1), lambda i: (i, 0)),
        ],
        out_specs=pl.BlockSpec((NBLK, OUT_F), lambda i: (i, 0)),
        out_shape=jax.ShapeDtypeStruct((N, OUT_F), jnp.float32),
    )(h, Wo, bo, mask)


def _edge_mlp_body(prea_ref, preb_ref, geom_ref, ea_ref, w1e_ref, b1_ref,
                   w2_ref, b2_ref, o_ref):
    pre = prea_ref[...] + preb_ref[...]
    ead = jnp.concatenate([geom_ref[:, :1], ea_ref[...]], axis=1)
    z = _silu(pre + ead @ w1e_ref[...] + b1_ref[...])
    o_ref[...] = _silu(z @ w2_ref[...] + b2_ref[...])


def _edge_mlp(preA, preB, geom0, ea, W1e, b1, W2, b2):
    return pl.pallas_call(
        _edge_mlp_body,
        grid=(E // EBLK,),
        in_specs=[
            pl.BlockSpec((EBLK, HID), lambda i: (i, 0)),
            pl.BlockSpec((EBLK, HID), lambda i: (i, 0)),
            pl.BlockSpec((EBLK, 4), lambda i: (i, 0)),
            pl.BlockSpec((EBLK, 16), lambda i: (i, 0)),
            _full((17, HID)), _full((HID,)),
            _full((HID, HID)), _full((HID,)),
        ],
        out_specs=pl.BlockSpec((EBLK, HID), lambda i: (i, 0)),
        out_shape=jax.ShapeDtypeStruct((E, HID), jnp.float32),
    )(preA, preB, geom0, ea, W1e, b1, W2, b2)


def _coord_mlp_body(prea_ref, preb_ref, geom0_ref, geomc_ref, ea_ref,
                    w1e_ref, b1_ref, w2_ref, b2_ref, w3_ref, o_ref):
    pre = prea_ref[...] + preb_ref[...]
    ead = jnp.concatenate([geom0_ref[:, :1], ea_ref[...]], axis=1)
    z = _silu(pre + ead @ w1e_ref[...] + b1_ref[...])
    z = _silu(z @ w2_ref[...] + b2_ref[...])
    phi = jnp.tanh(z @ w3_ref[...]) * COORDS_RANGE
    gc = geomc_ref[...]
    d = gc[:, 1:4]
    radial = gc[:, :1]
    scale = phi / (jnp.sqrt(radial + 1e-8) + NORM_CONST)
    o_ref[...] = jnp.concatenate(
        [d * scale, jnp.zeros((d.shape[0], 13), jnp.float32)], axis=1)


def _coord_mlp(preA, preB, geom0, geomc, ea, Wc1e, bc1, Wc2, bc2, Wc3):
    return pl.pallas_call(
        _coord_mlp_body,
        grid=(E // EBLK,),
        in_specs=[
            pl.BlockSpec((EBLK, HID), lambda i: (i, 0)),
            pl.BlockSpec((EBLK, HID), lambda i: (i, 0)),
            pl.BlockSpec((EBLK, 4), lambda i: (i, 0)),
            pl.BlockSpec((EBLK, 4), lambda i: (i, 0)),
            pl.BlockSpec((EBLK, 16), lambda i: (i, 0)),
            _full((17, HID)), _full((HID,)),
            _full((HID, HID)), _full((HID,)),
            _full((HID, 1)),
        ],
        out_specs=pl.BlockSpec((EBLK, 16), lambda i: (i, 0)),
        out_shape=jax.ShapeDtypeStruct((E, 16), jnp.float32),
    )(preA, preB, geom0, geomc, ea, Wc1e, bc1, Wc2, bc2, Wc3)


NW = 32
EPW = E // NW
CCH = 1000


def _sc_mesh():
    return plsc.VectorSubcoreMesh(core_axis_name="c", subcore_axis_name="s")


def _combine(A, B, row, col):
    @functools.partial(
        pl.kernel,
        out_type=(jax.ShapeDtypeStruct((E, HID), jnp.float32),
                  jax.ShapeDtypeStruct((E, HID), jnp.float32)),
        mesh=_sc_mesh(),
        scratch_types=[
            pltpu.VMEM((CCH,), jnp.int32),
            pltpu.VMEM((CCH,), jnp.int32),
            pltpu.VMEM((CCH, HID), jnp.float32),
            pltpu.VMEM((CCH, HID), jnp.float32),
            pltpu.SemaphoreType.DMA,
            pltpu.SemaphoreType.DMA,
        ],
        compiler_params=pltpu.CompilerParams(use_tc_tiling_on_sc=False),
    )
    def k(a_hbm, b_hbm, row_hbm, col_hbm, oa_hbm, ob_hbm,
          rid, cid, bufa, bufb, s1, s2):
        wid = lax.axis_index("s") * 2 + lax.axis_index("c")
        base = wid * EPW

        def body(j, carry):
            off = base + j * CCH
            pltpu.sync_copy(row_hbm.at[pl.ds(off, CCH)], rid)
            pltpu.sync_copy(col_hbm.at[pl.ds(off, CCH)], cid)
            ca = pltpu.async_copy(a_hbm.at[rid], bufa, s1)
            cb = pltpu.async_copy(b_hbm.at[cid], bufb, s2)
            ca.wait()
            cb.wait()
            pltpu.sync_copy(bufa, oa_hbm.at[pl.ds(off, CCH)])
            pltpu.sync_copy(bufb, ob_hbm.at[pl.ds(off, CCH)])
            return carry

        lax.fori_loop(0, EPW // CCH, body, 0)

    return k(A, B, row, col)


def _segsum(vals, row, K, zeros):
    npr = N // 16

    @functools.partial(
        pl.kernel,
        out_type=jax.ShapeDtypeStruct((2 * N, K), jnp.float32),
        mesh=_sc_mesh(),
        scratch_types=[
            pltpu.VMEM((CCH,), jnp.int32),
            pltpu.VMEM((CCH, K), jnp.float32),
            pltpu.VMEM_SHARED((N, K), jnp.float32),
        ],
        compiler_params=pltpu.CompilerParams(use_tc_tiling_on_sc=False),
    )
    def k(vals_hbm, row_hbm, zeros_hbm, out_hbm, rid, vbuf, acc):
        ci = lax.axis_index("c")
        sid = lax.axis_index("s")
        wid = sid * 2 + ci
        pltpu.sync_copy(zeros_hbm.at[pl.ds(sid * npr, npr)],
                        acc.at[pl.ds(sid * npr, npr)])
        plsc.subcore_barrier()
        base = wid * EPW

        def body(j, carry):
            off = base + j * CCH
            pltpu.sync_copy(row_hbm.at[pl.ds(off, CCH)], rid)
            pltpu.sync_copy(vals_hbm.at[pl.ds(off, CCH)], vbuf)
            pltpu.sync_copy(vbuf, acc.at[rid], add=True)
            return carry

        lax.fori_loop(0, EPW // CCH, body, 0)
        plsc.subcore_barrier()
        pltpu.sync_copy(acc.at[pl.ds(sid * npr, npr)],
                        out_hbm.at[pl.ds(ci * N + sid * npr, npr)])

    return k(vals, row, zeros)


def _sc_geom(px, py, pz, row, col):
    @functools.partial(
        pl.kernel,
        out_type=jax.ShapeDtypeStruct((E * 4,), jnp.float32),
        mesh=_sc_mesh(),
        scratch_types=[
            pltpu.VMEM((N,), jnp.float32),
            pltpu.VMEM((N,), jnp.float32),
            pltpu.VMEM((N,), jnp.float32),
            pltpu.VMEM((CCH,), jnp.int32),
            pltpu.VMEM((CCH,), jnp.int32),
            pltpu.VMEM((CCH * 4,), jnp.float32),
        ],
        compiler_params=pltpu.CompilerParams(
            use_tc_tiling_on_sc=False, needs_layout_passes=False),
    )
    def k(px_hbm, py_hbm, pz_hbm, row_hbm, col_hbm, out_hbm,
          pxv, pyv, pzv, rid, cid, gbuf):
        wid = lax.axis_index("s") * 2 + lax.axis_index("c")
        pltpu.sync_copy(px_hbm, pxv)
        pltpu.sync_copy(py_hbm, pyv)
        pltpu.sync_copy(pz_hbm, pzv)
        base = wid * EPW

        def chunk(j, carry):
            off = base + j * CCH
            pltpu.sync_copy(row_hbm.at[pl.ds(off, CCH)], rid)
            pltpu.sync_copy(col_hbm.at[pl.ds(off, CCH)], cid)

            def grp(g, carry2):
                rvec = rid[pl.ds(g * 16, 16)]
                cvec = cid[pl.ds(g * 16, 16)]
                dx = plsc.load_gather(pxv, [rvec]) - plsc.load_gather(pxv, [cvec])
                dy = plsc.load_gather(pyv, [rvec]) - plsc.load_gather(pyv, [cvec])
                dz = plsc.load_gather(pzv, [rvec]) - plsc.load_gather(pzv, [cvec])
                dist = dx * dx + dy * dy + dz * dz
                rows = g * 64 + lax.iota(jnp.int32, 16) * 4
                plsc.store_scatter(gbuf, [rows], dist)
                plsc.store_scatter(gbuf, [rows + 1], dx)
                plsc.store_scatter(gbuf, [rows + 2], dy)
                plsc.store_scatter(gbuf, [rows + 3], dz)
                return carry2

            lax.fori_loop(0, CCH // 16, grp, 0)
            pltpu.sync_copy(gbuf, out_hbm.at[pl.ds(off * 4, CCH * 4)])
            return carry

        lax.fori_loop(0, EPW // CCH, chunk, 0)

    return k(px, py, pz, row, col)


def _split_w1(W1):
    return W1[:HID], W1[HID:2 * HID], W1[2 * HID:]


def kernel(x, pos, mask, edge_attr, params, edge_index):
    row = edge_index[0]
    col = edge_index[1]
    blocks = params['blocks']

    geom0 = _sc_geom(pos[:, 0], pos[:, 1], pos[:, 2], row, col).reshape(E, 4)

    gcl0 = blocks[0]['gcls'][0]
    (W1_0, b1_0), (W2_0, b2_0) = gcl0['edge_mlp']
    W1r0, W1c0, W1e0 = _split_w1(W1_0)
    We, be = params['emb_in']
    h, A, B = _emb_in(x, We, be, W1r0, W1c0)
    zeros64 = jnp.zeros((N, HID), jnp.float32)
    zeros16 = jnp.zeros((N, 16), jnp.float32)

    for bi, blk in enumerate(blocks):
        gcl = blk['gcls'][0]
        (W1, b1), (W2, b2) = gcl['edge_mlp']
        W1r, W1c, W1e = _split_w1(W1)
        geomc = geom0 if bi == 0 else _sc_geom(
            pos[:, 0], pos[:, 1], pos[:, 2], row, col).reshape(E, 4)
        preA, preB = _combine(A, B, row, col)
        m = _edge_mlp(preA, preB, geom0, edge_attr, W1e, b1, W2, b2)
        s2n = _segsum(m, row, HID, zeros64)

        (Wn1, bn1), (Wn2, bn2) = gcl['node_mlp']
        Wn1h, Wn1a = Wn1[:HID], Wn1[HID:]
        (Wc1, bc1), (Wc2, bc2), (Wc3,) = blk['coord_mlp']
        Wc1r, Wc1c, Wc1e = _split_w1(Wc1)
        proj_ws = [Wc1r, Wc1c]
        if bi + 1 < len(blocks):
            gcl_n = blocks[bi + 1]['gcls'][0]
            W1n = gcl_n['edge_mlp'][0][0]
            W1rn, W1cn, _ = _split_w1(W1n)
            proj_ws += [W1rn, W1cn]
        outs = _node_update(h, s2n, mask, Wn1h, Wn1a, bn1, Wn2, bn2, proj_ws)
        h, Ac, Bc = outs[0], outs[1], outs[2]
        if bi + 1 < len(blocks):
            A, B = outs[3], outs[4]

        pcA, pcB = _combine(Ac, Bc, row, col)
        trans = _coord_mlp(pcA, pcB, geom0, geomc, edge_attr,
                           Wc1e, bc1, Wc2, bc2, Wc3)
        c2n = _segsum(trans, row, 16, zeros16)
        cagg = c2n[:N, :3] + c2n[N:, :3]
        pos = pos + (cagg * (1.0 / NORM_FACTOR)) * mask
        h = h * mask

    Wo, bo = params['emb_out']
    return (_emb_out(h, Wo, bo, mask), pos)


# speedup vs baseline: 3.5668x; 1.1895x over previous
import functools

import jax
import jax.numpy as jnp
from jax import lax
from jax.experimental import pallas as pl
from jax.experimental.pallas import tpu as pltpu
from jax.experimental.pallas import tpu_sc as plsc

N = 10000
E = 320000
HID = 64
IN_F = 128
OUT_F = 128
NORM_FACTOR = 100.0
NORM_CONST = 1.0
COORDS_RANGE = 15.0

NBLK = 1000
EBLK = 2560


def _silu(v):
    return v * jax.nn.sigmoid(v)


def _full(shape):
    return pl.BlockSpec(shape, lambda i: tuple(0 for _ in shape))


def _emb_in_body(x_ref, we_ref, be_ref, wr_ref, wc_ref, h_ref, a_ref, b_ref):
    h = x_ref[...] @ we_ref[...] + be_ref[...]
    h_ref[...] = h
    a_ref[...] = h @ wr_ref[...]
    b_ref[...] = h @ wc_ref[...]


def _emb_in(x, We, be, W1r, W1c):
    return pl.pallas_call(
        _emb_in_body,
        grid=(N // NBLK,),
        in_specs=[
            pl.BlockSpec((NBLK, IN_F), lambda i: (i, 0)),
            _full((IN_F, HID)), _full((HID,)),
            _full((HID, HID)), _full((HID, HID)),
        ],
        out_specs=[
            pl.BlockSpec((NBLK, HID), lambda i: (i, 0)),
            pl.BlockSpec((NBLK, HID), lambda i: (i, 0)),
            pl.BlockSpec((NBLK, HID), lambda i: (i, 0)),
        ],
        out_shape=[
            jax.ShapeDtypeStruct((N, HID), jnp.float32),
            jax.ShapeDtypeStruct((N, HID), jnp.float32),
            jax.ShapeDtypeStruct((N, HID), jnp.float32),
        ],
    )(x, We, be, W1r, W1c)


def _node_update_body(h_ref, s0_ref, s1_ref, mask_ref, wn1h_ref, wn1a_ref,
                      bn1_ref, wn2_ref, bn2_ref, *proj_refs):
    nproj = (len(proj_refs) - 1) // 2
    h = h_ref[...]
    agg = (s0_ref[...] + s1_ref[...]) * (1.0 / NORM_FACTOR)
    u = _silu(h @ wn1h_ref[...] + agg @ wn1a_ref[...] + bn1_ref[...])
    hn = (h + u @ wn2_ref[...] + bn2_ref[...]) * mask_ref[...]
    out_refs = proj_refs[nproj:]
    out_refs[0][...] = hn
    for k in range(nproj):
        out_refs[1 + k][...] = hn @ proj_refs[k][...]


def _node_update(h, s2n, mask, Wn1h, Wn1a, bn1, Wn2, bn2, proj_ws):
    nproj = len(proj_ws)
    return pl.pallas_call(
        _node_update_body,
        grid=(N // NBLK,),
        in_specs=[
            pl.BlockSpec((NBLK, HID), lambda i: (i, 0)),
            pl.BlockSpec((NBLK, HID), lambda i: (i, 0)),
            pl.BlockSpec((NBLK, HID), lambda i: (i + N // NBLK, 0)),
            pl.BlockSpec((NBLK, 1), lambda i: (i, 0)),
            _full((HID, HID)), _full((HID, HID)), _full((HID,)),
            _full((HID, HID)), _full((HID,)),
        ] + [_full((HID, HID)) for _ in range(nproj)],
        out_specs=[pl.BlockSpec((NBLK, HID), lambda i: (i, 0))
                   for _ in range(1 + nproj)],
        out_shape=[jax.ShapeDtypeStruct((N, HID), jnp.float32)
                   for _ in range(1 + nproj)],
    )(h, s2n, s2n, mask, Wn1h, Wn1a, bn1, Wn2, bn2, *proj_ws)


def _emb_out_body(h_ref, w_ref, b_ref, mask_ref, o_ref):
    o_ref[...] = (h_ref[...] @ w_ref[...] + b_ref[...]) * mask_ref[...]


def _emb_out(h, Wo, bo, mask):
    return pl.pallas_call(
        _emb_out_body,
        grid=(N // NBLK,),
        in_specs=[
            pl.BlockSpec((NBLK, HID), lambda i: (i, 0)),
            _full((HID, OUT_F)), _full((OUT_F,)),
            pl.BlockSpec((NBLK, 1), lambda i: (i, 0)),
        ],
        out_specs=pl.BlockSpec((NBLK, OUT_F), lambda i: (i, 0)),
        out_shape=jax.ShapeDtypeStruct((N, OUT_F), jnp.float32),
    )(h, Wo, bo, mask)


def _edge_mlp_body(pre_ref, geom_ref, ea_ref, w1e_ref, b1_ref,
                   w2_ref, b2_ref, o_ref):
    pre = pre_ref[...]
    ead = jnp.concatenate([geom_ref[:, :1], ea_ref[...]], axis=1)
    z = _silu(pre + ead @ w1e_ref[...] + b1_ref[...])
    o_ref[...] = _silu(z @ w2_ref[...] + b2_ref[...])


def _edge_mlp(pre, geom0, ea, W1e, b1, W2, b2):
    return pl.pallas_call(
        _edge_mlp_body,
        grid=(E // EBLK,),
        in_specs=[
            pl.BlockSpec((EBLK, HID), lambda i: (i, 0)),
            pl.BlockSpec((EBLK, 4), lambda i: (i, 0)),
            pl.BlockSpec((EBLK, 16), lambda i: (i, 0)),
            _full((17, HID)), _full((HID,)),
            _full((HID, HID)), _full((HID,)),
        ],
        out_specs=pl.BlockSpec((EBLK, HID), lambda i: (i, 0)),
        out_shape=jax.ShapeDtypeStruct((E, HID), jnp.float32),
    )(pre, geom0, ea, W1e, b1, W2, b2)


def _coord_mlp_body(pre_ref, geom0_ref, geomc_ref, ea_ref,
                    w1e_ref, b1_ref, w2_ref, b2_ref, w3_ref, o_ref):
    pre = pre_ref[...]
    ead = jnp.concatenate([geom0_ref[:, :1], ea_ref[...]], axis=1)
    z = _silu(pre + ead @ w1e_ref[...] + b1_ref[...])
    z = _silu(z @ w2_ref[...] + b2_ref[...])
    phi = jnp.tanh(z @ w3_ref[...]) * COORDS_RANGE
    gc = geomc_ref[...]
    d = gc[:, 1:4]
    radial = gc[:, :1]
    scale = phi / (jnp.sqrt(radial + 1e-8) + NORM_CONST)
    o_ref[...] = jnp.concatenate(
        [d * scale, jnp.zeros((d.shape[0], 13), jnp.float32)], axis=1)


def _coord_mlp(pre, geom0, geomc, ea, Wc1e, bc1, Wc2, bc2, Wc3):
    return pl.pallas_call(
        _coord_mlp_body,
        grid=(E // EBLK,),
        in_specs=[
            pl.BlockSpec((EBLK, HID), lambda i: (i, 0)),
            pl.BlockSpec((EBLK, 4), lambda i: (i, 0)),
            pl.BlockSpec((EBLK, 4), lambda i: (i, 0)),
            pl.BlockSpec((EBLK, 16), lambda i: (i, 0)),
            _full((17, HID)), _full((HID,)),
            _full((HID, HID)), _full((HID,)),
            _full((HID, 1)),
        ],
        out_specs=pl.BlockSpec((EBLK, 16), lambda i: (i, 0)),
        out_shape=jax.ShapeDtypeStruct((E, 16), jnp.float32),
    )(pre, geom0, geomc, ea, Wc1e, bc1, Wc2, bc2, Wc3)


NW = 32
EPW = E // NW
CCH = 1000


def _sc_mesh():
    return plsc.VectorSubcoreMesh(core_axis_name="c", subcore_axis_name="s")


CCB = 400
NPAIR = (EPW // CCB) // 2


def _combine(A, B, row, col, pxyz=None):
    with_geom = pxyz is not None
    out_type = [jax.ShapeDtypeStruct((E, HID), jnp.float32)]
    scratch = [
        pltpu.VMEM((CCB,), jnp.int32), pltpu.VMEM((CCB,), jnp.int32),
        pltpu.VMEM((CCB,), jnp.int32), pltpu.VMEM((CCB,), jnp.int32),
        pltpu.VMEM((CCB, HID), jnp.float32),
        pltpu.VMEM((CCB, HID), jnp.float32),
    ] + [pltpu.SemaphoreType.DMA] * 6
    if with_geom:
        out_type.append(jax.ShapeDtypeStruct((E * 4,), jnp.float32))
        scratch += [
            pltpu.VMEM((N,), jnp.float32), pltpu.VMEM((N,), jnp.float32),
            pltpu.VMEM((N,), jnp.float32),
            pltpu.VMEM((CCB * 4,), jnp.float32),
            pltpu.VMEM((CCB * 4,), jnp.float32),
        ]

    @functools.partial(
        pl.kernel,
        out_type=tuple(out_type) if with_geom else out_type[0],
        mesh=_sc_mesh(),
        scratch_types=scratch,
        compiler_params=pltpu.CompilerParams(
            use_tc_tiling_on_sc=False, needs_layout_passes=False),
    )
    def k(*refs):
        if with_geom:
            (a_hbm, b_hbm, row_hbm, col_hbm, px_hbm, py_hbm, pz_hbm,
             o_hbm, g_hbm,
             rid0, cid0, rid1, cid1, buf0, buf1,
             sa0, sa1, sb0, sb1, sw0, sw1,
             pxv, pyv, pzv, gbuf0, gbuf1) = refs
        else:
            (a_hbm, b_hbm, row_hbm, col_hbm, o_hbm,
             rid0, cid0, rid1, cid1, buf0, buf1,
             sa0, sa1, sb0, sb1, sw0, sw1) = refs
        wid = lax.axis_index("s") * 2 + lax.axis_index("c")
        base = wid * EPW
        if with_geom:
            pltpu.sync_copy(px_hbm, pxv)
            pltpu.sync_copy(py_hbm, pyv)
            pltpu.sync_copy(pz_hbm, pzv)

        def geom_chunk(rid, cid, gbuf):
            def grp(g, carry2):
                rvec = rid[pl.ds(g * 16, 16)]
                cvec = cid[pl.ds(g * 16, 16)]
                dx = plsc.load_gather(pxv, [rvec]) - plsc.load_gather(pxv, [cvec])
                dy = plsc.load_gather(pyv, [rvec]) - plsc.load_gather(pyv, [cvec])
                dz = plsc.load_gather(pzv, [rvec]) - plsc.load_gather(pzv, [cvec])
                dist = dx * dx + dy * dy + dz * dz
                rows = g * 64 + lax.iota(jnp.int32, 16) * 4
                plsc.store_scatter(gbuf, [rows], dist)
                plsc.store_scatter(gbuf, [rows + 1], dx)
                plsc.store_scatter(gbuf, [rows + 2], dy)
                plsc.store_scatter(gbuf, [rows + 3], dz)
                return carry2

            lax.fori_loop(0, CCB // 16, grp, 0)

        def do_chunk_sync(off, rid, cid, buf, gbuf, sa, sb, sw):
            pltpu.sync_copy(row_hbm.at[pl.ds(off, CCB)], rid)
            pltpu.sync_copy(col_hbm.at[pl.ds(off, CCB)], cid)
            pltpu.async_copy(a_hbm.at[rid], buf, sa).wait()
            cb = pltpu.async_copy(b_hbm.at[cid], buf, sb, add=True)
            if with_geom:
                geom_chunk(rid, cid, gbuf)
            cb.wait()
            w = pltpu.async_copy(buf, o_hbm.at[pl.ds(off, CCB)], sw)
            if with_geom:
                pltpu.sync_copy(gbuf, g_hbm.at[pl.ds(off * 4, CCB * 4)])
            w.wait()

        def pair(p, carry):
            j0 = p * 2
            off0 = base + j0 * CCB
            off1 = off0 + CCB
            pltpu.sync_copy(row_hbm.at[pl.ds(off0, CCB)], rid0)
            pltpu.sync_copy(col_hbm.at[pl.ds(off0, CCB)], cid0)
            pltpu.sync_copy(row_hbm.at[pl.ds(off1, CCB)], rid1)
            pltpu.sync_copy(col_hbm.at[pl.ds(off1, CCB)], cid1)
            a0 = pltpu.async_copy(a_hbm.at[rid0], buf0, sa0)
            a1 = pltpu.async_copy(a_hbm.at[rid1], buf1, sa1)
            a0.wait()
            b0 = pltpu.async_copy(b_hbm.at[cid0], buf0, sb0, add=True)
            a1.wait()
            b1 = pltpu.async_copy(b_hbm.at[cid1], buf1, sb1, add=True)
            if with_geom:
                geom_chunk(rid0, cid0, gbuf0)
            b0.wait()
            w0 = pltpu.async_copy(buf0, o_hbm.at[pl.ds(off0, CCB)], sw0)
            if with_geom:
                pltpu.sync_copy(gbuf0, g_hbm.at[pl.ds(off0 * 4, CCB * 4)])
                geom_chunk(rid1, cid1, gbuf1)
            b1.wait()
            w1 = pltpu.async_copy(buf1, o_hbm.at[pl.ds(off1, CCB)], sw1)
            if with_geom:
                pltpu.sync_copy(gbuf1, g_hbm.at[pl.ds(off1 * 4, CCB * 4)])
            w0.wait()
            w1.wait()
            return carry

        lax.fori_loop(0, NPAIR, pair, 0)
        do_chunk_sync(base + 2 * NPAIR * CCB, rid0, cid0, buf0,
                      gbuf0 if with_geom else None, sa0, sb0, sw0)

    if with_geom:
        return k(A, B, row, col, *pxyz)
    return k(A, B, row, col)


def _segsum(vals, row, K, zeros):
    npr = N // 16

    @functools.partial(
        pl.kernel,
        out_type=jax.ShapeDtypeStruct((2 * N, K), jnp.float32),
        mesh=_sc_mesh(),
        scratch_types=[
            pltpu.VMEM((CCH,), jnp.int32),
            pltpu.VMEM((CCH, K), jnp.float32),
            pltpu.VMEM_SHARED((N, K), jnp.float32),
        ],
        compiler_params=pltpu.CompilerParams(use_tc_tiling_on_sc=False),
    )
    def k(vals_hbm, row_hbm, zeros_hbm, out_hbm, rid, vbuf, acc):
        ci = lax.axis_index("c")
        sid = lax.axis_index("s")
        wid = sid * 2 + ci
        pltpu.sync_copy(zeros_hbm.at[pl.ds(sid * npr, npr)],
                        acc.at[pl.ds(sid * npr, npr)])
        plsc.subcore_barrier()
        base = wid * EPW

        def body(j, carry):
            off = base + j * CCH
            pltpu.sync_copy(row_hbm.at[pl.ds(off, CCH)], rid)
            pltpu.sync_copy(vals_hbm.at[pl.ds(off, CCH)], vbuf)
            pltpu.sync_copy(vbuf, acc.at[rid], add=True)
            return carry

        lax.fori_loop(0, EPW // CCH, body, 0)
        plsc.subcore_barrier()
        pltpu.sync_copy(acc.at[pl.ds(sid * npr, npr)],
                        out_hbm.at[pl.ds(ci * N + sid * npr, npr)])

    return k(vals, row, zeros)


def _split_w1(W1):
    return W1[:HID], W1[HID:2 * HID], W1[2 * HID:]


def kernel(x, pos, mask, edge_attr, params, edge_index):
    row = edge_index[0]
    col = edge_index[1]
    blocks = params['blocks']

    gcl0 = blocks[0]['gcls'][0]
    (W1_0, b1_0), (W2_0, b2_0) = gcl0['edge_mlp']
    W1r0, W1c0, W1e0 = _split_w1(W1_0)
    We, be = params['emb_in']
    h, A, B = _emb_in(x, We, be, W1r0, W1c0)
    zeros64 = jnp.zeros((N, HID), jnp.float32)
    zeros16 = jnp.zeros((N, 16), jnp.float32)

    for bi, blk in enumerate(blocks):
        gcl = blk['gcls'][0]
        (W1, b1), (W2, b2) = gcl['edge_mlp']
        W1r, W1c, W1e = _split_w1(W1)
        pre, geomf = _combine(A, B, row, col,
                              (pos[:, 0], pos[:, 1], pos[:, 2]))
        geomc = geomf.reshape(E, 4)
        if bi == 0:
            geom0 = geomc
        m = _edge_mlp(pre, geom0, edge_attr, W1e, b1, W2, b2)
        s2n = _segsum(m, row, HID, zeros64)

        (Wn1, bn1), (Wn2, bn2) = gcl['node_mlp']
        Wn1h, Wn1a = Wn1[:HID], Wn1[HID:]
        (Wc1, bc1), (Wc2, bc2), (Wc3,) = blk['coord_mlp']
        Wc1r, Wc1c, Wc1e = _split_w1(Wc1)
        proj_ws = [Wc1r, Wc1c]
        if bi + 1 < len(blocks):
            gcl_n = blocks[bi + 1]['gcls'][0]
            W1n = gcl_n['edge_mlp'][0][0]
            W1rn, W1cn, _ = _split_w1(W1n)
            proj_ws += [W1rn, W1cn]
        outs = _node_update(h, s2n, mask, Wn1h, Wn1a, bn1, Wn2, bn2, proj_ws)
        h, Ac, Bc = outs[0], outs[1], outs[2]
        if bi + 1 < len(blocks):
            A, B = outs[3], outs[4]

        pre_c = _combine(Ac, Bc, row, col)
        trans = _coord_mlp(pre_c, geom0, geomc, edge_attr,
                           Wc1e, bc1, Wc2, bc2, Wc3)
        c2n = _segsum(trans, row, 16, zeros16)
        cagg = c2n[:N, :3] + c2n[N:, :3]
        pos = pos + (cagg * (1.0 / NORM_FACTOR)) * mask
        h = h * mask

    Wo, bo = params['emb_out']
    return (_emb_out(h, Wo, bo, mask), pos)
